# Initial kernel scaffold; baseline (speedup 1.0000x reference)
import jax
import jax.numpy as jnp
from jax.experimental import pallas as pl


def kernel(node_node_edge_index, batch_rows, batch_cols, input_features, input_feature_s, W_1, W_2, W_3, lin_w, lin_b):
    raise NotImplementedError("write your pallas kernel here")


# speedup vs baseline: 4.0257x; 4.0257x over previous
import functools

import jax
import jax.numpy as jnp
from jax import lax
from jax.experimental import pallas as pl
from jax.experimental.pallas import tpu as pltpu
from jax.experimental.pallas import tpu_sc as plsc

N = 10000
E = 320000
BN = 1024
EB = 32768
D = 128
DEPTH = 2

NW = 32
NSUB = 16
CH = 79
CHB = 8
NACC = 10112
ZROWS = 632
DUMP = 10008


def _sc_body(h_hbm, srcc, dstc, bcolc, browc, zeros_hbm, part_nv, part_s,
             acc, acc_s, idx_src, idx_dst, idx_bc, idx_br, rows, sem):
    c = lax.axis_index("c")
    s = lax.axis_index("s")
    w = c * NSUB + s

    pltpu.sync_copy(srcc.at[w], idx_src)
    pltpu.sync_copy(dstc.at[w], idx_dst)
    pltpu.sync_copy(bcolc.at[w], idx_bc)
    pltpu.sync_copy(browc.at[w], idx_br)

    pltpu.sync_copy(zeros_hbm, acc.at[pl.ds(s * ZROWS, ZROWS)])
    pltpu.sync_copy(zeros_hbm.at[pl.ds(0, BN // NSUB)],
                    acc_s.at[pl.ds(s * (BN // NSUB), BN // NSUB)])
    plsc.subcore_barrier()

    def nn_step(j, carry):
        pltpu.async_copy(h_hbm.at[idx_src.at[j]], rows, sem).wait()
        pltpu.sync_copy(rows, acc.at[idx_dst.at[j]], add=True)
        return carry

    lax.fori_loop(0, CH, nn_step, 0)

    def b_step(j, carry):
        pltpu.async_copy(h_hbm.at[idx_bc.at[j]], rows, sem).wait()
        pltpu.sync_copy(rows, acc_s.at[idx_br.at[j]], add=True)
        return carry

    lax.fori_loop(0, CHB, b_step, 0)
    plsc.subcore_barrier()

    pltpu.sync_copy(acc.at[pl.ds(s * ZROWS, ZROWS)],
                    part_nv.at[c, pl.ds(s * ZROWS, ZROWS)])
    pltpu.sync_copy(acc_s.at[pl.ds(s * (BN // NSUB), BN // NSUB)],
                    part_s.at[c, pl.ds(s * (BN // NSUB), BN // NSUB)])


@functools.cache
def _sc_segsum():
    return pl.kernel(
        _sc_body,
        mesh=plsc.VectorSubcoreMesh(core_axis_name="c", subcore_axis_name="s"),
        out_type=[
            jax.ShapeDtypeStruct((2, NACC, D), jnp.float32),
            jax.ShapeDtypeStruct((2, BN, D), jnp.float32),
        ],
        scratch_types=[
            pltpu.VMEM_SHARED((NACC, D), jnp.float32),
            pltpu.VMEM_SHARED((BN, D), jnp.float32),
            pltpu.VMEM((CH, 128), jnp.int32),
            pltpu.VMEM((CH, 128), jnp.int32),
            pltpu.VMEM((CHB, 128), jnp.int32),
            pltpu.VMEM((CHB, 128), jnp.int32),
            pltpu.VMEM((128, D), jnp.float32),
            pltpu.SemaphoreType.DMA,
        ],
    )


def _normalize(x):
    n = jnp.sqrt(jnp.sum(x * x, axis=1, keepdims=True))
    return x / jnp.maximum(n, 1e-12)


def _init_body(x_ref, xs_ref, w1_ref, h_ref, hs_ref):
    w1 = w1_ref[...]
    h = jnp.maximum(jnp.dot(x_ref[...], w1, preferred_element_type=jnp.float32), 0.0)
    h_ref[...] = _normalize(h)
    hs = jnp.maximum(jnp.dot(xs_ref[...], w1, preferred_element_type=jnp.float32), 0.0)
    hs_ref[...] = _normalize(hs)


def _dense(h, hn, w2, w3, la, lb, bias):
    z = lax.dot_general(jnp.dot(h, w2, preferred_element_type=jnp.float32), la,
                        (((1,), (1,)), ((), ())), preferred_element_type=jnp.float32)
    z = z + lax.dot_general(jnp.dot(hn, w3, preferred_element_type=jnp.float32), lb,
                            (((1,), (1,)), ((), ())), preferred_element_type=jnp.float32)
    return _normalize(jnp.maximum(z + bias, 0.0))


def _layer_node_body(h_ref, pn_ref, w2_ref, w3_ref, la_ref, lb_ref, b_ref, ho_ref):
    hn = pn_ref[0] + pn_ref[1]
    ho_ref[...] = _dense(h_ref[...], hn, w2_ref[...], w3_ref[...],
                         la_ref[...], lb_ref[...], b_ref[...])


def _layer_batch_body(hs_ref, ps_ref, w2_ref, w3_ref, la_ref, lb_ref, b_ref, hso_ref):
    hn = ps_ref[0] + ps_ref[1]
    hso_ref[...] = _dense(hs_ref[...], hn, w2_ref[...], w3_ref[...],
                          la_ref[...], lb_ref[...], b_ref[...])


_BLK = 1000

_tc_init = pl.pallas_call(
    _init_body,
    out_shape=[jax.ShapeDtypeStruct((N, D), jnp.float32),
               jax.ShapeDtypeStruct((BN, D), jnp.float32)],
)

_w_spec = pl.BlockSpec((D, D), lambda i: (0, 0))
_b_spec = pl.BlockSpec((1, D), lambda i: (0, 0))

_tc_layer_node = pl.pallas_call(
    _layer_node_body,
    grid=(N // _BLK,),
    in_specs=[
        pl.BlockSpec((_BLK, D), lambda i: (i, 0)),
        pl.BlockSpec((2, _BLK, D), lambda i: (0, i, 0)),
        _w_spec, _w_spec, _w_spec, _w_spec, _b_spec,
    ],
    out_specs=pl.BlockSpec((_BLK, D), lambda i: (i, 0)),
    out_shape=jax.ShapeDtypeStruct((N, D), jnp.float32),
)

_tc_layer_batch = pl.pallas_call(
    _layer_batch_body,
    out_shape=jax.ShapeDtypeStruct((BN, D), jnp.float32),
)


def kernel(node_node_edge_index, batch_rows, batch_cols, input_features,
           input_feature_s, W_1, W_2, W_3, lin_w, lin_b):
    dst = node_node_edge_index[0].astype(jnp.int32)
    src = node_node_edge_index[1].astype(jnp.int32)
    brows = batch_rows.astype(jnp.int32)
    bcols = batch_cols.astype(jnp.int32)

    pad = NW * CH * 128 - E
    srcc = jnp.concatenate([src, jnp.zeros((pad,), jnp.int32)]).reshape(NW, CH, 128)
    dstc = jnp.concatenate([dst, jnp.full((pad,), DUMP, jnp.int32)]).reshape(NW, CH, 128)
    bcolc = bcols.reshape(NW, CHB, 128)
    browc = brows.reshape(NW, CHB, 128)

    zeros_hbm = jnp.zeros((ZROWS, D), jnp.float32)
    lin_a = lin_w[:, :D]
    lin_bm = lin_w[:, D:]
    bias = lin_b.reshape(1, D)

    h, hs = _tc_init(input_features, input_feature_s, W_1)
    for _ in range(DEPTH):
        part_nv, part_s = _sc_segsum()(h, srcc, dstc, bcolc, browc, zeros_hbm)
        h = _tc_layer_node(h, part_nv[:, :N], W_2, W_3, lin_a, lin_bm, bias)
        hs = _tc_layer_batch(hs, part_s, W_2, W_3, lin_a, lin_bm, bias)
    return (h, hs)
